# Initial kernel scaffold; baseline (speedup 1.0000x reference)
import jax
import jax.numpy as jnp
from jax.experimental import pallas as pl


def kernel(last_problem, last_skill, last_ans, next_problem, next_skill, next_ans, pro_embed, skill_embed, ans_embed, time_embed, ls_state, pro_state0, skill_state0, akt_pro_diff, akt_pro_change, W_out1, b_out1, W_out2, b_out2, W_pf, b_pf, W_ps, b_ps, W_af, b_af, W_sf, b_sf, W_ss, b_ss, W_as, b_as):
    raise NotImplementedError("write your pallas kernel here")


# speedup vs baseline: 26.7520x; 26.7520x over previous
import functools

import jax
import jax.numpy as jnp
from jax import lax
from jax.experimental import pallas as pl
from jax.experimental.pallas import tpu as pltpu
from jax.experimental.pallas import tpu_sc as plsc

D = 128
S = 50
B = 1024
N = B * S

_NC = 2
_NS = 16
_NW = _NC * _NS
_BPW = N // _NW
_CH = 400
_NCHUNK = _BPW // _CH


def _sc_gather_all(pro_embed, skill_embed, change, diff_mat, np_idx, np_hi,
                   ns_idx):
    mesh = plsc.VectorSubcoreMesh(core_axis_name="c", subcore_axis_name="s")

    @functools.partial(
        pl.kernel,
        mesh=mesh,
        out_type=(
            jax.ShapeDtypeStruct((N, D), jnp.float32),
            jax.ShapeDtypeStruct((N, D), jnp.float32),
            jax.ShapeDtypeStruct((N, D), jnp.float32),
            jax.ShapeDtypeStruct((N, D), jnp.float32),
        ),
        scratch_types=[
            pltpu.VMEM((_BPW,), jnp.int32),
            pltpu.VMEM((_CH, D), jnp.float32),
            pltpu.SemaphoreType.DMA,
        ],
    )
    def k(pro_hbm, skill_hbm, change_hbm, diff_hbm, npi_hbm, nphi_hbm, nsi_hbm,
          pro_out, skill_out, change_out, diff_out, idx_v, rows_v, sem):
        wid = lax.axis_index("s") * _NC + lax.axis_index("c")
        base = wid * _BPW

        pltpu.sync_copy(npi_hbm.at[pl.ds(base, _BPW)], idx_v)
        for ci in range(_NCHUNK):
            off = ci * _CH
            idx_c = idx_v.at[pl.ds(off, _CH)]
            pltpu.async_copy(pro_hbm.at[idx_c], rows_v, sem).wait()
            pltpu.sync_copy(rows_v, pro_out.at[pl.ds(base + off, _CH)])

        pltpu.sync_copy(nphi_hbm.at[pl.ds(base, _BPW)], idx_v)
        for ci in range(_NCHUNK):
            off = ci * _CH
            idx_c = idx_v.at[pl.ds(off, _CH)]
            pltpu.async_copy(diff_hbm.at[idx_c], rows_v, sem).wait()
            pltpu.sync_copy(rows_v, diff_out.at[pl.ds(base + off, _CH)])

        pltpu.sync_copy(nsi_hbm.at[pl.ds(base, _BPW)], idx_v)
        for ci in range(_NCHUNK):
            off = ci * _CH
            idx_c = idx_v.at[pl.ds(off, _CH)]
            pltpu.async_copy(skill_hbm.at[idx_c], rows_v, sem).wait()
            pltpu.sync_copy(rows_v, skill_out.at[pl.ds(base + off, _CH)])
            pltpu.async_copy(change_hbm.at[idx_c], rows_v, sem).wait()
            pltpu.sync_copy(rows_v, change_out.at[pl.ds(base + off, _CH)])

    return k(pro_embed, skill_embed, change, diff_mat, np_idx, np_hi, ns_idx)


_BB = 64
_NB = B // _BB
_TPAD = 56


def _scan_kernel(pro_ref, skill_ref, change_ref, diffrow_ref, lo_ref, na_ref,
                 np_ref, ns_ref, ae_ref, te_ref, ls_ref, ps0_ref, ss0_ref,
                 wpf_ref, bpf_ref, wsf_ref, bsf_ref, waf_ref, baf_ref,
                 wps_ref, bps_ref, wss_ref, bss_ref, was_ref, bas_ref,
                 w1_ref, b1_ref, w2_ref, b2_ref,
                 out_ref, histp_ref, hists_ref):
    f32 = jnp.float32
    dot = functools.partial(jnp.dot, preferred_element_type=f32)

    np_all = np_ref[...]
    ns_all = ns_ref[...]
    a0 = ae_ref[0:1, :]
    a1 = ae_ref[1:2, :]

    wpf = wpf_ref[...]
    wsf = wsf_ref[...]
    waf = waf_ref[...]
    wps = wps_ref[...]
    wss = wss_ref[...]
    was = was_ref[...]
    w1 = w1_ref[...]

    te56 = te_ref[0:_TPAD, :]
    tp_tab = dot(te56, wpf[D:, :])
    ts_tab = dot(te56, wsf[D:, :])
    cst_af = dot(te_ref[1:2, :], waf[D:, :]) + baf_ref[...]

    jj3 = lax.broadcasted_iota(jnp.int32, (S, _BB, 1), 0)
    lane_t = lax.broadcasted_iota(jnp.int32, (_BB, _TPAD), 1)
    lane_d = lax.broadcasted_iota(jnp.int32, (_BB, D), 1)

    histp_ref[0, :, :] = jnp.broadcast_to(ps0_ref[...], (_BB, D))
    hists_ref[0, :, :] = jnp.broadcast_to(ss0_ref[...], (_BB, D))

    alls0 = jnp.broadcast_to(ls_ref[...], (_BB, D))
    pacc0 = jnp.zeros((_BB, _TPAD), f32)

    def body(t, carry):
        alls, pacc = carry

        np_row = np_ref[pl.ds(t, 1)]
        ns_row = ns_ref[pl.ds(t, 1)]
        written = jj3 < t
        eqp = (np_all == np_row) & written
        eqs = (ns_all == ns_row) & written
        lbpt3 = jnp.max(jnp.where(eqp, jj3, 0), axis=0, keepdims=True)
        lbst3 = jnp.max(jnp.where(eqs, jj3, 0), axis=0, keepdims=True)

        maskp = jj3 == lbpt3
        masks = jj3 == lbst3
        lbps = jnp.sum(jnp.where(maskp, histp_ref[...], 0.0), axis=0)
        lbss = jnp.sum(jnp.where(masks, hists_ref[...], 0.0), axis=0)

        ohp = (lane_t == (t - lbpt3[0])).astype(f32)
        ohs = (lane_t == (t - lbst3[0])).astype(f32)
        gp_b = dot(ohp, tp_tab)
        gs_b = dot(ohs, ts_tab)

        lbps = lbps * jax.nn.sigmoid(dot(lbps, wpf[:D, :]) + gp_b + bpf_ref[...])
        lbss = lbss * jax.nn.sigmoid(dot(lbss, wsf[:D, :]) + gs_b + bsf_ref[...])
        lbas = alls * jax.nn.sigmoid(dot(alls, waf[:D, :]) + cst_af)

        pro_t = pro_ref[pl.ds(t, 1)][0]
        skill_t = skill_ref[pl.ds(t, 1)][0]
        change_t = change_ref[pl.ds(t, 1)][0]
        diffrow_t = diffrow_ref[pl.ds(t, 1)][0]
        lo_t = lo_ref[pl.ds(t, 1)][0]
        diff_t = jnp.sum(jnp.where(lane_d == lo_t, diffrow_t, 0.0),
                         axis=1, keepdims=True)
        na_t = na_ref[pl.ds(t, 1)][0]
        npe = pro_t + skill_t + diff_t * change_t
        nx = npe + a0 + na_t * (a1 - a0)

        h = jax.nn.relu(dot(lbas, w1[0:D, :]) + dot(lbps, w1[D:2 * D, :])
                        + dot(lbss, w1[2 * D:3 * D, :]) + dot(npe, w1[3 * D:, :])
                        + b1_ref[...])
        logit = jnp.sum(h * w2_ref[...], axis=1, keepdims=True) + b2_ref[...]
        p = jax.nn.sigmoid(logit)
        oh_t = (lane_t == t).astype(f32)
        pacc = pacc + p * oh_t

        alls_new = lbas + jnp.tanh(dot(lbas, was[:D, :]) + dot(nx, was[D:, :])
                                   + bas_ref[...])
        ips = lbps + jnp.tanh(dot(lbps, wps[:D, :]) + dot(nx, wps[D:, :])
                              + bps_ref[...])
        iss = lbss + jnp.tanh(dot(lbss, wss[:D, :]) + dot(nx, wss[D:, :])
                              + bss_ref[...])
        histp_ref[pl.ds(t, 1)] = ips[None]
        hists_ref[pl.ds(t, 1)] = iss[None]
        return alls_new, pacc

    _, pacc = lax.fori_loop(0, S, body, (alls0, pacc0))
    out_ref[...] = pacc[:, :S]


def _run_scan(pro_sm, skill_sm, change_sm, diffrow_sm, lo_sm, na_sm, np_sm,
              ns_sm, ans_embed, time_embed, ls_state, ps0, ss0, weights):
    (wpf, bpf, wsf, bsf, waf, baf, wps, bps, wss, bss, was, bas,
     w1, b1, w2, b2) = weights
    row3 = pl.BlockSpec((S, _BB, D), lambda i: (0, i, 0))
    row1 = pl.BlockSpec((S, _BB, 1), lambda i: (0, i, 0))

    def full(a):
        return pl.BlockSpec(a.shape, lambda i: tuple(0 for _ in a.shape))

    consts = [ans_embed, time_embed, ls_state, ps0, ss0,
              wpf, bpf, wsf, bsf, waf, baf, wps, bps, wss, bss, was, bas,
              w1, b1, w2, b2]
    return pl.pallas_call(
        _scan_kernel,
        grid=(_NB,),
        in_specs=[row3, row3, row3, row3, row1, row1, row1, row1] + [full(c) for c in consts],
        out_specs=pl.BlockSpec((_BB, S), lambda i: (i, 0)),
        out_shape=jax.ShapeDtypeStruct((B, S), jnp.float32),
        scratch_shapes=[pltpu.VMEM((S, _BB, D), jnp.float32),
                        pltpu.VMEM((S, _BB, D), jnp.float32)],
        compiler_params=pltpu.CompilerParams(
            dimension_semantics=("arbitrary",),
            vmem_limit_bytes=63 * 1024 * 1024),
    )(pro_sm, skill_sm, change_sm, diffrow_sm, lo_sm, na_sm, np_sm, ns_sm,
      *consts)


def kernel(last_problem, last_skill, last_ans, next_problem, next_skill,
           next_ans, pro_embed, skill_embed, ans_embed, time_embed, ls_state,
           pro_state0, skill_state0, akt_pro_diff, akt_pro_change, W_out1,
           b_out1, W_out2, b_out2, W_pf, b_pf, W_ps, b_ps, W_af, b_af, W_sf,
           b_sf, W_ss, b_ss, W_as, b_as):
    npb = next_problem.reshape(last_problem.shape)
    nsb = next_skill.reshape(last_skill.shape)
    nab = next_ans.reshape(last_ans.shape)

    np_idx = npb.T.reshape(-1)
    ns_idx = nsb.T.reshape(-1)

    diff_mat = jnp.concatenate(
        [akt_pro_diff[:, 0], jnp.zeros((96,), jnp.float32)]).reshape(782, D)
    np_hi = lax.shift_right_logical(np_idx, 7)
    np_lo = lax.bitwise_and(np_idx, 127)

    pro_rows, skill_rows, change_rows, diff_rows = _sc_gather_all(
        pro_embed, skill_embed, akt_pro_change, diff_mat, np_idx, np_hi, ns_idx)

    pro_sm = pro_rows.reshape(S, B, D)
    skill_sm = skill_rows.reshape(S, B, D)
    change_sm = change_rows.reshape(S, B, D)
    diffrow_sm = diff_rows.reshape(S, B, D)
    lo_sm = np_lo.reshape(S, B, 1)
    na_sm = nab.T.reshape(S, B, 1).astype(jnp.float32)
    np_sm = npb.T.reshape(S, B, 1)
    ns_sm = nsb.T.reshape(S, B, 1)

    weights = (W_pf, b_pf.reshape(1, D), W_sf, b_sf.reshape(1, D),
               W_af, b_af.reshape(1, D), W_ps, b_ps.reshape(1, D),
               W_ss, b_ss.reshape(1, D), W_as, b_as.reshape(1, D),
               W_out1, b_out1.reshape(1, D), W_out2.reshape(1, D),
               b_out2.reshape(1, 1))
    return _run_scan(pro_sm, skill_sm, change_sm, diffrow_sm, lo_sm, na_sm,
                     np_sm, ns_sm, ans_embed, time_embed, ls_state,
                     pro_state0[0:1], skill_state0[0:1], weights)


# speedup vs baseline: 83.8382x; 3.1339x over previous
import functools

import jax
import jax.numpy as jnp
from jax import lax
from jax.experimental import pallas as pl
from jax.experimental.pallas import tpu as pltpu
from jax.experimental.pallas import tpu_sc as plsc

D = 128
S = 50
B = 1024
N = B * S

_NC = 2
_NS = 16
_NW = _NC * _NS
_BPW = N // _NW
_CH = 400
_NCHUNK = _BPW // _CH


def _sc_gather_all(pro_embed, skill_embed, change, diff_mat, np_idx, np_hi,
                   ns_idx):
    mesh = plsc.VectorSubcoreMesh(core_axis_name="c", subcore_axis_name="s")

    @functools.partial(
        pl.kernel,
        mesh=mesh,
        out_type=(
            jax.ShapeDtypeStruct((N, D), jnp.float32),
            jax.ShapeDtypeStruct((N, D), jnp.float32),
            jax.ShapeDtypeStruct((N, D), jnp.float32),
            jax.ShapeDtypeStruct((N, D), jnp.float32),
        ),
        scratch_types=[
            pltpu.VMEM((_BPW,), jnp.int32),
            pltpu.VMEM((_CH, D), jnp.float32),
            pltpu.SemaphoreType.DMA,
        ],
    )
    def k(pro_hbm, skill_hbm, change_hbm, diff_hbm, npi_hbm, nphi_hbm, nsi_hbm,
          pro_out, skill_out, change_out, diff_out, idx_v, rows_v, sem):
        wid = lax.axis_index("s") * _NC + lax.axis_index("c")
        base = wid * _BPW

        pltpu.sync_copy(npi_hbm.at[pl.ds(base, _BPW)], idx_v)
        for ci in range(_NCHUNK):
            off = ci * _CH
            idx_c = idx_v.at[pl.ds(off, _CH)]
            pltpu.async_copy(pro_hbm.at[idx_c], rows_v, sem).wait()
            pltpu.sync_copy(rows_v, pro_out.at[pl.ds(base + off, _CH)])

        pltpu.sync_copy(nphi_hbm.at[pl.ds(base, _BPW)], idx_v)
        for ci in range(_NCHUNK):
            off = ci * _CH
            idx_c = idx_v.at[pl.ds(off, _CH)]
            pltpu.async_copy(diff_hbm.at[idx_c], rows_v, sem).wait()
            pltpu.sync_copy(rows_v, diff_out.at[pl.ds(base + off, _CH)])

        pltpu.sync_copy(nsi_hbm.at[pl.ds(base, _BPW)], idx_v)
        for ci in range(_NCHUNK):
            off = ci * _CH
            idx_c = idx_v.at[pl.ds(off, _CH)]
            pltpu.async_copy(skill_hbm.at[idx_c], rows_v, sem).wait()
            pltpu.sync_copy(rows_v, skill_out.at[pl.ds(base + off, _CH)])
            pltpu.async_copy(change_hbm.at[idx_c], rows_v, sem).wait()
            pltpu.sync_copy(rows_v, change_out.at[pl.ds(base + off, _CH)])

    return k(pro_embed, skill_embed, change, diff_mat, np_idx, np_hi, ns_idx)


_BB = 128
_NB = B // _BB
_TPAD = 56


def _scan_kernel(pro_ref, skill_ref, change_ref, diffrow_ref, lo_ref, na_ref,
                 np_ref, ns_ref, aet_ref, tet_ref, lst_ref, ps0t_ref, ss0t_ref,
                 wpfa_ref, wpfb_ref, bpf_ref, wsfa_ref, wsfb_ref, bsf_ref,
                 wafa_ref, wafb_ref, baf_ref, wpsa_ref, wpsb_ref, bps_ref,
                 wssa_ref, wssb_ref, bss_ref, wasa_ref, wasb_ref, bas_ref,
                 w1a_ref, w1b_ref, w1c_ref, w1d_ref, b1_ref, w2_ref, b2_ref,
                 out_ref, histp_ref, hists_ref, lbp_scr, lbs_scr, pacc_scr):
    f32 = jnp.float32
    i32 = jnp.int32
    dot = functools.partial(jnp.dot, preferred_element_type=f32)

    np_all = np_ref[0]
    ns_all = ns_ref[0]

    jjj = lax.broadcasted_iota(i32, (S, S, _BB), 0)
    ttt = lax.broadcasted_iota(i32, (S, S, _BB), 1)
    eqp = (np_all[:, None, :] == np_all[None, :, :]) & (jjj < ttt)
    eqs = (ns_all[:, None, :] == ns_all[None, :, :]) & (jjj < ttt)
    lbp_scr[...] = jnp.max(jnp.where(eqp, jjj, 0), axis=0)
    lbs_scr[...] = jnp.max(jnp.where(eqs, jjj, 0), axis=0)

    tp_tab = dot(wpfb_ref[...], tet_ref[:, 0:_TPAD])
    ts_tab = dot(wsfb_ref[...], tet_ref[:, 0:_TPAD])
    caf = dot(wafb_ref[...], tet_ref[:, 1:2]) + baf_ref[...]

    a0 = aet_ref[:, 0:1]
    a1 = aet_ref[:, 1:2]

    jj_s1b = lax.broadcasted_iota(i32, (S, 1, _BB), 0)
    sub56 = lax.broadcasted_iota(i32, (_TPAD, _BB), 0)
    subd = lax.broadcasted_iota(i32, (D, _BB), 0)

    histp_ref[0] = jnp.broadcast_to(ps0t_ref[...], (D, _BB))
    hists_ref[0] = jnp.broadcast_to(ss0t_ref[...], (D, _BB))
    alls0 = jnp.broadcast_to(lst_ref[...], (D, _BB))

    def body(t, alls):
        lbpt_row = lbp_scr[pl.ds(t, 1)]
        lbst_row = lbs_scr[pl.ds(t, 1)]
        maskp = jj_s1b == lbpt_row
        masks = jj_s1b == lbst_row
        lbps = jnp.sum(jnp.where(maskp, histp_ref[...], 0.0), axis=0)
        lbss = jnp.sum(jnp.where(masks, hists_ref[...], 0.0), axis=0)

        ohp = (sub56 == (t - lbpt_row)).astype(f32)
        ohs = (sub56 == (t - lbst_row)).astype(f32)

        lbps = lbps * jax.nn.sigmoid(
            dot(wpfa_ref[...], lbps) + dot(tp_tab, ohp) + bpf_ref[...])
        lbss = lbss * jax.nn.sigmoid(
            dot(wsfa_ref[...], lbss) + dot(ts_tab, ohs) + bsf_ref[...])
        lbas = alls * jax.nn.sigmoid(dot(wafa_ref[...], alls) + caf)

        pro_t = jnp.transpose(pro_ref[pl.ds(t, 1)][0])
        skill_t = jnp.transpose(skill_ref[pl.ds(t, 1)][0])
        change_t = jnp.transpose(change_ref[pl.ds(t, 1)][0])
        drow_t = jnp.transpose(diffrow_ref[pl.ds(t, 1)][0])
        lo_row = lo_ref[pl.ds(0, 1), pl.ds(t, 1), :][0]
        diff_row = jnp.sum(jnp.where(subd == lo_row, drow_t, 0.0),
                           axis=0, keepdims=True)
        na_row = na_ref[pl.ds(0, 1), pl.ds(t, 1), :][0]
        npe = pro_t + skill_t + diff_row * change_t
        nx = npe + a0 + na_row * (a1 - a0)

        h = jax.nn.relu(dot(w1a_ref[...], lbas) + dot(w1b_ref[...], lbps)
                        + dot(w1c_ref[...], lbss) + dot(w1d_ref[...], npe)
                        + b1_ref[...])
        logit = jnp.sum(h * w2_ref[...], axis=0, keepdims=True) + b2_ref[...]
        pacc_scr[pl.ds(t, 1)] = jax.nn.sigmoid(logit)

        alls_new = lbas + jnp.tanh(
            dot(wasa_ref[...], lbas) + dot(wasb_ref[...], nx) + bas_ref[...])
        ips = lbps + jnp.tanh(
            dot(wpsa_ref[...], lbps) + dot(wpsb_ref[...], nx) + bps_ref[...])
        iss = lbss + jnp.tanh(
            dot(wssa_ref[...], lbss) + dot(wssb_ref[...], nx) + bss_ref[...])
        histp_ref[pl.ds(t, 1)] = ips[None]
        hists_ref[pl.ds(t, 1)] = iss[None]
        return alls_new

    lax.fori_loop(0, S, body, alls0)
    out_ref[0] = pacc_scr[...]


def _run_scan(pro_sm, skill_sm, change_sm, diffrow_sm, lo_r, na_r, np_r,
              ns_r, consts):
    row3 = pl.BlockSpec((S, _BB, D), lambda i: (0, i, 0))
    rowp = pl.BlockSpec((1, S, _BB), lambda i: (i, 0, 0))

    def full(a):
        return pl.BlockSpec(a.shape, lambda i: tuple(0 for _ in a.shape))

    return pl.pallas_call(
        _scan_kernel,
        grid=(_NB,),
        in_specs=[row3, row3, row3, row3, rowp, rowp, rowp, rowp]
                 + [full(c) for c in consts],
        out_specs=pl.BlockSpec((1, S, _BB), lambda i: (i, 0, 0)),
        out_shape=jax.ShapeDtypeStruct((_NB, S, _BB), jnp.float32),
        scratch_shapes=[pltpu.VMEM((S, D, _BB), jnp.float32),
                        pltpu.VMEM((S, D, _BB), jnp.float32),
                        pltpu.VMEM((S, _BB), jnp.int32),
                        pltpu.VMEM((S, _BB), jnp.int32),
                        pltpu.VMEM((S, _BB), jnp.float32)],
        compiler_params=pltpu.CompilerParams(
            dimension_semantics=("arbitrary",),
            vmem_limit_bytes=63 * 1024 * 1024),
    )(pro_sm, skill_sm, change_sm, diffrow_sm, lo_r, na_r, np_r, ns_r, *consts)


def _plane(arr_bs):
    return arr_bs.T.reshape(S, _NB, _BB).transpose(1, 0, 2)


def kernel(last_problem, last_skill, last_ans, next_problem, next_skill,
           next_ans, pro_embed, skill_embed, ans_embed, time_embed, ls_state,
           pro_state0, skill_state0, akt_pro_diff, akt_pro_change, W_out1,
           b_out1, W_out2, b_out2, W_pf, b_pf, W_ps, b_ps, W_af, b_af, W_sf,
           b_sf, W_ss, b_ss, W_as, b_as):
    npb = next_problem.reshape(last_problem.shape)
    nsb = next_skill.reshape(last_skill.shape)
    nab = next_ans.reshape(last_ans.shape)

    np_idx = npb.T.reshape(-1)
    ns_idx = nsb.T.reshape(-1)

    diff_mat = jnp.concatenate(
        [akt_pro_diff[:, 0], jnp.zeros((96,), jnp.float32)]).reshape(782, D)
    np_hi = lax.shift_right_logical(np_idx, 7)
    np_lo = lax.bitwise_and(np_idx, 127)

    pro_rows, skill_rows, change_rows, diff_rows = _sc_gather_all(
        pro_embed, skill_embed, akt_pro_change, diff_mat, np_idx, np_hi, ns_idx)

    pro_sm = pro_rows.reshape(S, B, D)
    skill_sm = skill_rows.reshape(S, B, D)
    change_sm = change_rows.reshape(S, B, D)
    diffrow_sm = diff_rows.reshape(S, B, D)
    lo_r = _plane(np_lo.reshape(S, B).T)
    na_r = _plane(nab).astype(jnp.float32)
    np_r = _plane(npb)
    ns_r = _plane(nsb)

    def tT(w):
        return jnp.transpose(w)

    consts = [
        tT(ans_embed), tT(time_embed), tT(ls_state),
        tT(pro_state0[0:1]), tT(skill_state0[0:1]),
        tT(W_pf[:D]), tT(W_pf[D:]), b_pf.reshape(D, 1),
        tT(W_sf[:D]), tT(W_sf[D:]), b_sf.reshape(D, 1),
        tT(W_af[:D]), tT(W_af[D:]), b_af.reshape(D, 1),
        tT(W_ps[:D]), tT(W_ps[D:]), b_ps.reshape(D, 1),
        tT(W_ss[:D]), tT(W_ss[D:]), b_ss.reshape(D, 1),
        tT(W_as[:D]), tT(W_as[D:]), b_as.reshape(D, 1),
        tT(W_out1[0:D]), tT(W_out1[D:2 * D]), tT(W_out1[2 * D:3 * D]),
        tT(W_out1[3 * D:]), b_out1.reshape(D, 1), W_out2, b_out2.reshape(1, 1),
    ]
    out = _run_scan(pro_sm, skill_sm, change_sm, diffrow_sm, lo_r, na_r,
                    np_r, ns_r, consts)
    return out.transpose(0, 2, 1).reshape(B, S)


# speedup vs baseline: 106.5403x; 1.2708x over previous
import functools

import jax
import jax.numpy as jnp
from jax import lax
from jax.experimental import pallas as pl
from jax.experimental.pallas import tpu as pltpu
from jax.experimental.pallas import tpu_sc as plsc

D = 128
S = 50
B = 1024
N = B * S

_NC = 2
_NS = 16
_NW = _NC * _NS
_BPW = N // _NW
_CH = 400
_NCHUNK = _BPW // _CH


def _sc_gather_all(pro_embed, skill_embed, change, diff_mat, np_idx, np_hi,
                   ns_idx):
    mesh = plsc.VectorSubcoreMesh(core_axis_name="c", subcore_axis_name="s")

    @functools.partial(
        pl.kernel,
        mesh=mesh,
        out_type=(
            jax.ShapeDtypeStruct((N, D), jnp.float32),
            jax.ShapeDtypeStruct((N, D), jnp.float32),
            jax.ShapeDtypeStruct((N, D), jnp.float32),
            jax.ShapeDtypeStruct((N, D), jnp.float32),
        ),
        scratch_types=[
            pltpu.VMEM((_BPW,), jnp.int32),
            pltpu.VMEM((_CH, D), jnp.float32),
            pltpu.SemaphoreType.DMA,
        ],
    )
    def k(pro_hbm, skill_hbm, change_hbm, diff_hbm, npi_hbm, nphi_hbm, nsi_hbm,
          pro_out, skill_out, change_out, diff_out, idx_v, rows_v, sem):
        wid = lax.axis_index("s") * _NC + lax.axis_index("c")
        base = wid * _BPW

        pltpu.sync_copy(npi_hbm.at[pl.ds(base, _BPW)], idx_v)
        for ci in range(_NCHUNK):
            off = ci * _CH
            idx_c = idx_v.at[pl.ds(off, _CH)]
            pltpu.async_copy(pro_hbm.at[idx_c], rows_v, sem).wait()
            pltpu.sync_copy(rows_v, pro_out.at[pl.ds(base + off, _CH)])

        pltpu.sync_copy(nphi_hbm.at[pl.ds(base, _BPW)], idx_v)
        for ci in range(_NCHUNK):
            off = ci * _CH
            idx_c = idx_v.at[pl.ds(off, _CH)]
            pltpu.async_copy(diff_hbm.at[idx_c], rows_v, sem).wait()
            pltpu.sync_copy(rows_v, diff_out.at[pl.ds(base + off, _CH)])

        pltpu.sync_copy(nsi_hbm.at[pl.ds(base, _BPW)], idx_v)
        for ci in range(_NCHUNK):
            off = ci * _CH
            idx_c = idx_v.at[pl.ds(off, _CH)]
            pltpu.async_copy(skill_hbm.at[idx_c], rows_v, sem).wait()
            pltpu.sync_copy(rows_v, skill_out.at[pl.ds(base + off, _CH)])
            pltpu.async_copy(change_hbm.at[idx_c], rows_v, sem).wait()
            pltpu.sync_copy(rows_v, change_out.at[pl.ds(base + off, _CH)])

    return k(pro_embed, skill_embed, change, diff_mat, np_idx, np_hi, ns_idx)


_BB = 128
_NB = B // _BB
_TPAD = 56


def _scan_kernel(pro_ref, skill_ref, change_ref, diffrow_ref, lo_ref, na_ref,
                 np_ref, ns_ref, aet_ref, tet_ref, lst_ref, ps0t_ref, ss0t_ref,
                 wpfa_ref, wpfb_ref, bpf_ref, wsfa_ref, wsfb_ref, bsf_ref,
                 wafa_ref, wafb_ref, baf_ref, wpsa_ref, wpsb_ref, bps_ref,
                 wssa_ref, wssb_ref, bss_ref, wasa_ref, wasb_ref, bas_ref,
                 w1a_ref, w1b_ref, w1c_ref, w1d_ref, b1_ref, w2_ref, b2_ref,
                 out_ref, histp_ref, hists_ref, lbp_scr, lbs_scr, pacc_scr):
    f32 = jnp.float32
    i32 = jnp.int32
    dot = functools.partial(jnp.dot, preferred_element_type=f32)

    np_all = np_ref[0]
    ns_all = ns_ref[0]

    jjj = lax.broadcasted_iota(i32, (S, S, _BB), 0)
    ttt = lax.broadcasted_iota(i32, (S, S, _BB), 1)
    eqp = (np_all[:, None, :] == np_all[None, :, :]) & (jjj < ttt)
    eqs = (ns_all[:, None, :] == ns_all[None, :, :]) & (jjj < ttt)
    lbp_scr[...] = jnp.max(jnp.where(eqp, jjj, 0), axis=0)
    lbs_scr[...] = jnp.max(jnp.where(eqs, jjj, 0), axis=0)

    tp_tab = dot(wpfb_ref[...], tet_ref[:, 0:_TPAD])
    ts_tab = dot(wsfb_ref[...], tet_ref[:, 0:_TPAD])
    caf = dot(wafb_ref[...], tet_ref[:, 1:2]) + baf_ref[...]

    a0 = aet_ref[:, 0:1]
    a1 = aet_ref[:, 1:2]

    jj_s1b = lax.broadcasted_iota(i32, (S, 1, _BB), 0)
    sub56 = lax.broadcasted_iota(i32, (_TPAD, _BB), 0)
    subd = lax.broadcasted_iota(i32, (D, _BB), 0)

    histp_ref[...] = jnp.zeros((S, D, _BB), f32)
    hists_ref[...] = jnp.zeros((S, D, _BB), f32)
    histp_ref[0] = jnp.broadcast_to(ps0t_ref[...], (D, _BB))
    hists_ref[0] = jnp.broadcast_to(ss0t_ref[...], (D, _BB))
    alls0 = jnp.broadcast_to(lst_ref[...], (D, _BB))

    def body(t, alls, jmax):
        lbpt_row = lbp_scr[pl.ds(t, 1)]
        lbst_row = lbs_scr[pl.ds(t, 1)]
        maskpf = (jj_s1b[0:jmax] == lbpt_row).astype(f32)
        masksf = (jj_s1b[0:jmax] == lbst_row).astype(f32)
        lbps = jnp.sum(histp_ref[0:jmax] * maskpf, axis=0)
        lbss = jnp.sum(hists_ref[0:jmax] * masksf, axis=0)

        ohp = (sub56 == (t - lbpt_row)).astype(f32)
        ohs = (sub56 == (t - lbst_row)).astype(f32)

        lbps = lbps * jax.nn.sigmoid(
            dot(wpfa_ref[...], lbps) + dot(tp_tab, ohp) + bpf_ref[...])
        lbss = lbss * jax.nn.sigmoid(
            dot(wsfa_ref[...], lbss) + dot(ts_tab, ohs) + bsf_ref[...])
        lbas = alls * jax.nn.sigmoid(dot(wafa_ref[...], alls) + caf)

        pro_t = jnp.transpose(pro_ref[pl.ds(t, 1)][0])
        skill_t = jnp.transpose(skill_ref[pl.ds(t, 1)][0])
        change_t = jnp.transpose(change_ref[pl.ds(t, 1)][0])
        drow_t = jnp.transpose(diffrow_ref[pl.ds(t, 1)][0])
        lo_row = lo_ref[pl.ds(0, 1), pl.ds(t, 1), :][0]
        diff_row = jnp.sum(jnp.where(subd == lo_row, drow_t, 0.0),
                           axis=0, keepdims=True)
        na_row = na_ref[pl.ds(0, 1), pl.ds(t, 1), :][0]
        npe = pro_t + skill_t + diff_row * change_t
        nx = npe + a0 + na_row * (a1 - a0)

        h = jax.nn.relu(dot(w1a_ref[...], lbas) + dot(w1b_ref[...], lbps)
                        + dot(w1c_ref[...], lbss) + dot(w1d_ref[...], npe)
                        + b1_ref[...])
        logit = jnp.sum(h * w2_ref[...], axis=0, keepdims=True) + b2_ref[...]
        pacc_scr[pl.ds(t, 1)] = jax.nn.sigmoid(logit)

        alls_new = lbas + jnp.tanh(
            dot(wasa_ref[...], lbas) + dot(wasb_ref[...], nx) + bas_ref[...])
        ips = lbps + jnp.tanh(
            dot(wpsa_ref[...], lbps) + dot(wpsb_ref[...], nx) + bps_ref[...])
        iss = lbss + jnp.tanh(
            dot(wssa_ref[...], lbss) + dot(wssb_ref[...], nx) + bss_ref[...])
        histp_ref[pl.ds(t, 1)] = ips[None]
        hists_ref[pl.ds(t, 1)] = iss[None]
        return alls_new

    alls = alls0
    for seg_lo, seg_hi in ((0, 8), (8, 16), (16, 24), (24, 32), (32, 40),
                           (40, S)):
        alls = lax.fori_loop(seg_lo, seg_hi,
                             functools.partial(body, jmax=seg_hi), alls)
    out_ref[0] = pacc_scr[...]


def _run_scan(pro_sm, skill_sm, change_sm, diffrow_sm, lo_r, na_r, np_r,
              ns_r, consts):
    row3 = pl.BlockSpec((S, _BB, D), lambda i: (0, i, 0))
    rowp = pl.BlockSpec((1, S, _BB), lambda i: (i, 0, 0))

    def full(a):
        return pl.BlockSpec(a.shape, lambda i: tuple(0 for _ in a.shape))

    return pl.pallas_call(
        _scan_kernel,
        grid=(_NB,),
        in_specs=[row3, row3, row3, row3, rowp, rowp, rowp, rowp]
                 + [full(c) for c in consts],
        out_specs=pl.BlockSpec((1, S, _BB), lambda i: (i, 0, 0)),
        out_shape=jax.ShapeDtypeStruct((_NB, S, _BB), jnp.float32),
        scratch_shapes=[pltpu.VMEM((S, D, _BB), jnp.float32),
                        pltpu.VMEM((S, D, _BB), jnp.float32),
                        pltpu.VMEM((S, _BB), jnp.int32),
                        pltpu.VMEM((S, _BB), jnp.int32),
                        pltpu.VMEM((S, _BB), jnp.float32)],
        compiler_params=pltpu.CompilerParams(
            dimension_semantics=("arbitrary",),
            vmem_limit_bytes=63 * 1024 * 1024),
    )(pro_sm, skill_sm, change_sm, diffrow_sm, lo_r, na_r, np_r, ns_r, *consts)


def _plane(arr_bs):
    return arr_bs.T.reshape(S, _NB, _BB).transpose(1, 0, 2)


def kernel(last_problem, last_skill, last_ans, next_problem, next_skill,
           next_ans, pro_embed, skill_embed, ans_embed, time_embed, ls_state,
           pro_state0, skill_state0, akt_pro_diff, akt_pro_change, W_out1,
           b_out1, W_out2, b_out2, W_pf, b_pf, W_ps, b_ps, W_af, b_af, W_sf,
           b_sf, W_ss, b_ss, W_as, b_as):
    npb = next_problem.reshape(last_problem.shape)
    nsb = next_skill.reshape(last_skill.shape)
    nab = next_ans.reshape(last_ans.shape)

    np_idx = npb.T.reshape(-1)
    ns_idx = nsb.T.reshape(-1)

    diff_mat = jnp.concatenate(
        [akt_pro_diff[:, 0], jnp.zeros((96,), jnp.float32)]).reshape(782, D)
    np_hi = lax.shift_right_logical(np_idx, 7)
    np_lo = lax.bitwise_and(np_idx, 127)

    pro_rows, skill_rows, change_rows, diff_rows = _sc_gather_all(
        pro_embed, skill_embed, akt_pro_change, diff_mat, np_idx, np_hi, ns_idx)

    pro_sm = pro_rows.reshape(S, B, D)
    skill_sm = skill_rows.reshape(S, B, D)
    change_sm = change_rows.reshape(S, B, D)
    diffrow_sm = diff_rows.reshape(S, B, D)
    lo_r = _plane(np_lo.reshape(S, B).T)
    na_r = _plane(nab).astype(jnp.float32)
    np_r = _plane(npb)
    ns_r = _plane(nsb)

    def tT(w):
        return jnp.transpose(w)

    consts = [
        tT(ans_embed), tT(time_embed), tT(ls_state),
        tT(pro_state0[0:1]), tT(skill_state0[0:1]),
        tT(W_pf[:D]), tT(W_pf[D:]), b_pf.reshape(D, 1),
        tT(W_sf[:D]), tT(W_sf[D:]), b_sf.reshape(D, 1),
        tT(W_af[:D]), tT(W_af[D:]), b_af.reshape(D, 1),
        tT(W_ps[:D]), tT(W_ps[D:]), b_ps.reshape(D, 1),
        tT(W_ss[:D]), tT(W_ss[D:]), b_ss.reshape(D, 1),
        tT(W_as[:D]), tT(W_as[D:]), b_as.reshape(D, 1),
        tT(W_out1[0:D]), tT(W_out1[D:2 * D]), tT(W_out1[2 * D:3 * D]),
        tT(W_out1[3 * D:]), b_out1.reshape(D, 1), W_out2, b_out2.reshape(1, 1),
    ]
    out = _run_scan(pro_sm, skill_sm, change_sm, diffrow_sm, lo_r, na_r,
                    np_r, ns_r, consts)
    return out.transpose(0, 2, 1).reshape(B, S)


# speedup vs baseline: 106.8800x; 1.0032x over previous
import functools

import jax
import jax.numpy as jnp
from jax import lax
from jax.experimental import pallas as pl
from jax.experimental.pallas import tpu as pltpu
from jax.experimental.pallas import tpu_sc as plsc

D = 128
S = 50
B = 1024
N = B * S

_NC = 2
_NS = 16
_NW = _NC * _NS
_BPW = N // _NW
_CH = 400
_NCHUNK = _BPW // _CH


def _sc_gather_all(pro_embed, skill_embed, change, diff_mat, np_idx, np_hi,
                   ns_idx):
    mesh = plsc.VectorSubcoreMesh(core_axis_name="c", subcore_axis_name="s")

    @functools.partial(
        pl.kernel,
        mesh=mesh,
        out_type=(
            jax.ShapeDtypeStruct((N, D), jnp.float32),
            jax.ShapeDtypeStruct((N, D), jnp.float32),
            jax.ShapeDtypeStruct((N, D), jnp.float32),
            jax.ShapeDtypeStruct((N, D), jnp.float32),
        ),
        scratch_types=[
            pltpu.VMEM((_BPW,), jnp.int32),
            pltpu.VMEM((_BPW,), jnp.int32),
            pltpu.VMEM((_CH, D), jnp.float32),
            pltpu.VMEM((_CH, D), jnp.float32),
            pltpu.SemaphoreType.DMA,
            pltpu.SemaphoreType.DMA,
        ],
    )
    def k(pro_hbm, skill_hbm, change_hbm, diff_hbm, npi_hbm, nphi_hbm, nsi_hbm,
          pro_out, skill_out, change_out, diff_out, idxp_v, idxs_v,
          rows0_v, rows1_v, sem0, sem1):
        wid = lax.axis_index("s") * _NC + lax.axis_index("c")
        base = wid * _BPW

        pltpu.sync_copy(npi_hbm.at[pl.ds(base, _BPW)], idxp_v)
        pltpu.sync_copy(nsi_hbm.at[pl.ds(base, _BPW)], idxs_v)

        work = []
        for ci in range(_NCHUNK):
            work.append((pro_hbm, idxp_v, pro_out, ci))
            work.append((skill_hbm, idxs_v, skill_out, ci))
            work.append((change_hbm, idxs_v, change_out, ci))
        bufs = (rows0_v, rows1_v)
        sems = (sem0, sem1)

        def start(i):
            tbl, idx, _, ci = work[i]
            pltpu.async_copy(tbl.at[idx.at[pl.ds(ci * _CH, _CH)]],
                             bufs[i % 2], sems[i % 2])

        start(0)
        for i in range(len(work)):
            if i + 1 < len(work):
                start(i + 1)
            tbl, idx, out, ci = work[i]
            pltpu.make_async_copy(tbl.at[idx.at[pl.ds(ci * _CH, _CH)]],
                                  bufs[i % 2], sems[i % 2]).wait()
            pltpu.sync_copy(bufs[i % 2], out.at[pl.ds(base + ci * _CH, _CH)])

        pltpu.sync_copy(nphi_hbm.at[pl.ds(base, _BPW)], idxp_v)
        for ci in range(_NCHUNK):
            off = ci * _CH
            idx_c = idxp_v.at[pl.ds(off, _CH)]
            pltpu.async_copy(diff_hbm.at[idx_c], bufs[ci % 2], sems[ci % 2])
            if ci > 0:
                poff = (ci - 1) * _CH
                pltpu.make_async_copy(
                    diff_hbm.at[idxp_v.at[pl.ds(poff, _CH)]],
                    bufs[(ci - 1) % 2], sems[(ci - 1) % 2]).wait()
                pltpu.sync_copy(bufs[(ci - 1) % 2],
                                diff_out.at[pl.ds(base + poff, _CH)])
        last = _NCHUNK - 1
        pltpu.make_async_copy(diff_hbm.at[idxp_v.at[pl.ds(last * _CH, _CH)]],
                              bufs[last % 2], sems[last % 2]).wait()
        pltpu.sync_copy(bufs[last % 2],
                        diff_out.at[pl.ds(base + last * _CH, _CH)])

    return k(pro_embed, skill_embed, change, diff_mat, np_idx, np_hi, ns_idx)


_BB = 128
_NB = B // _BB
_TPAD = 56


def _scan_kernel(pro_ref, skill_ref, change_ref, diffrow_ref, lo_ref, na_ref,
                 np_ref, ns_ref, aet_ref, tet_ref, lst_ref, ps0t_ref, ss0t_ref,
                 wpfa_ref, wpfb_ref, bpf_ref, wsfa_ref, wsfb_ref, bsf_ref,
                 wafa_ref, wafb_ref, baf_ref, wpsa_ref, wpsb_ref, bps_ref,
                 wssa_ref, wssb_ref, bss_ref, wasa_ref, wasb_ref, bas_ref,
                 w1a_ref, w1b_ref, w1c_ref, w1d_ref, b1_ref, w2_ref, b2_ref,
                 out_ref, histp_ref, hists_ref, lbp_scr, lbs_scr, pacc_scr):
    f32 = jnp.float32
    i32 = jnp.int32
    dot = functools.partial(jnp.dot, preferred_element_type=f32)

    np_all = np_ref[0]
    ns_all = ns_ref[0]

    jjj = lax.broadcasted_iota(i32, (S, S, _BB), 0)
    ttt = lax.broadcasted_iota(i32, (S, S, _BB), 1)
    eqp = (np_all[:, None, :] == np_all[None, :, :]) & (jjj < ttt)
    eqs = (ns_all[:, None, :] == ns_all[None, :, :]) & (jjj < ttt)
    lbp_scr[...] = jnp.max(jnp.where(eqp, jjj, 0), axis=0)
    lbs_scr[...] = jnp.max(jnp.where(eqs, jjj, 0), axis=0)

    tp_tab = dot(wpfb_ref[...], tet_ref[:, 0:_TPAD])
    ts_tab = dot(wsfb_ref[...], tet_ref[:, 0:_TPAD])
    caf = dot(wafb_ref[...], tet_ref[:, 1:2]) + baf_ref[...]

    a0 = aet_ref[:, 0:1]
    a1 = aet_ref[:, 1:2]

    jj_s1b = lax.broadcasted_iota(i32, (S, 1, _BB), 0)
    sub56 = lax.broadcasted_iota(i32, (_TPAD, _BB), 0)
    subd = lax.broadcasted_iota(i32, (D, _BB), 0)

    histp_ref[...] = jnp.zeros((S, D, _BB), f32)
    hists_ref[...] = jnp.zeros((S, D, _BB), f32)
    histp_ref[0] = jnp.broadcast_to(ps0t_ref[...], (D, _BB))
    hists_ref[0] = jnp.broadcast_to(ss0t_ref[...], (D, _BB))
    alls0 = jnp.broadcast_to(lst_ref[...], (D, _BB))

    def body(t, alls, jmax):
        lbpt_row = lbp_scr[pl.ds(t, 1)]
        lbst_row = lbs_scr[pl.ds(t, 1)]
        maskpf = (jj_s1b[0:jmax] == lbpt_row).astype(f32)
        masksf = (jj_s1b[0:jmax] == lbst_row).astype(f32)
        lbps = jnp.sum(histp_ref[0:jmax] * maskpf, axis=0)
        lbss = jnp.sum(hists_ref[0:jmax] * masksf, axis=0)

        ohp = (sub56 == (t - lbpt_row)).astype(f32)
        ohs = (sub56 == (t - lbst_row)).astype(f32)

        lbps = lbps * jax.nn.sigmoid(
            dot(wpfa_ref[...], lbps) + dot(tp_tab, ohp) + bpf_ref[...])
        lbss = lbss * jax.nn.sigmoid(
            dot(wsfa_ref[...], lbss) + dot(ts_tab, ohs) + bsf_ref[...])
        lbas = alls * jax.nn.sigmoid(dot(wafa_ref[...], alls) + caf)

        pro_t = jnp.transpose(pro_ref[pl.ds(t, 1)][0])
        skill_t = jnp.transpose(skill_ref[pl.ds(t, 1)][0])
        change_t = jnp.transpose(change_ref[pl.ds(t, 1)][0])
        drow_t = jnp.transpose(diffrow_ref[pl.ds(t, 1)][0])
        lo_row = lo_ref[pl.ds(0, 1), pl.ds(t, 1), :][0]
        diff_row = jnp.sum(jnp.where(subd == lo_row, drow_t, 0.0),
                           axis=0, keepdims=True)
        na_row = na_ref[pl.ds(0, 1), pl.ds(t, 1), :][0]
        npe = pro_t + skill_t + diff_row * change_t
        nx = npe + a0 + na_row * (a1 - a0)

        h = jax.nn.relu(dot(w1a_ref[...], lbas) + dot(w1b_ref[...], lbps)
                        + dot(w1c_ref[...], lbss) + dot(w1d_ref[...], npe)
                        + b1_ref[...])
        logit = jnp.sum(h * w2_ref[...], axis=0, keepdims=True) + b2_ref[...]
        pacc_scr[pl.ds(t, 1)] = jax.nn.sigmoid(logit)

        alls_new = lbas + jnp.tanh(
            dot(wasa_ref[...], lbas) + dot(wasb_ref[...], nx) + bas_ref[...])
        ips = lbps + jnp.tanh(
            dot(wpsa_ref[...], lbps) + dot(wpsb_ref[...], nx) + bps_ref[...])
        iss = lbss + jnp.tanh(
            dot(wssa_ref[...], lbss) + dot(wssb_ref[...], nx) + bss_ref[...])
        histp_ref[pl.ds(t, 1)] = ips[None]
        hists_ref[pl.ds(t, 1)] = iss[None]
        return alls_new

    alls = alls0
    for seg_lo, seg_hi in ((0, 8), (8, 16), (16, 24), (24, 32), (32, 40),
                           (40, S)):
        alls = lax.fori_loop(seg_lo, seg_hi,
                             functools.partial(body, jmax=seg_hi), alls)
    out_ref[0] = pacc_scr[...]


def _run_scan(pro_sm, skill_sm, change_sm, diffrow_sm, lo_r, na_r, np_r,
              ns_r, consts):
    row3 = pl.BlockSpec((S, _BB, D), lambda i: (0, i, 0))
    rowp = pl.BlockSpec((1, S, _BB), lambda i: (i, 0, 0))

    def full(a):
        return pl.BlockSpec(a.shape, lambda i: tuple(0 for _ in a.shape))

    return pl.pallas_call(
        _scan_kernel,
        grid=(_NB,),
        in_specs=[row3, row3, row3, row3, rowp, rowp, rowp, rowp]
                 + [full(c) for c in consts],
        out_specs=pl.BlockSpec((1, S, _BB), lambda i: (i, 0, 0)),
        out_shape=jax.ShapeDtypeStruct((_NB, S, _BB), jnp.float32),
        scratch_shapes=[pltpu.VMEM((S, D, _BB), jnp.float32),
                        pltpu.VMEM((S, D, _BB), jnp.float32),
                        pltpu.VMEM((S, _BB), jnp.int32),
                        pltpu.VMEM((S, _BB), jnp.int32),
                        pltpu.VMEM((S, _BB), jnp.float32)],
        compiler_params=pltpu.CompilerParams(
            dimension_semantics=("arbitrary",),
            vmem_limit_bytes=63 * 1024 * 1024),
    )(pro_sm, skill_sm, change_sm, diffrow_sm, lo_r, na_r, np_r, ns_r, *consts)


def _plane(arr_bs):
    return arr_bs.T.reshape(S, _NB, _BB).transpose(1, 0, 2)


def kernel(last_problem, last_skill, last_ans, next_problem, next_skill,
           next_ans, pro_embed, skill_embed, ans_embed, time_embed, ls_state,
           pro_state0, skill_state0, akt_pro_diff, akt_pro_change, W_out1,
           b_out1, W_out2, b_out2, W_pf, b_pf, W_ps, b_ps, W_af, b_af, W_sf,
           b_sf, W_ss, b_ss, W_as, b_as):
    npb = next_problem.reshape(last_problem.shape)
    nsb = next_skill.reshape(last_skill.shape)
    nab = next_ans.reshape(last_ans.shape)

    np_idx = npb.T.reshape(-1)
    ns_idx = nsb.T.reshape(-1)

    diff_mat = jnp.concatenate(
        [akt_pro_diff[:, 0], jnp.zeros((96,), jnp.float32)]).reshape(782, D)
    np_hi = lax.shift_right_logical(np_idx, 7)
    np_lo = lax.bitwise_and(np_idx, 127)

    pro_rows, skill_rows, change_rows, diff_rows = _sc_gather_all(
        pro_embed, skill_embed, akt_pro_change, diff_mat, np_idx, np_hi, ns_idx)

    pro_sm = pro_rows.reshape(S, B, D)
    skill_sm = skill_rows.reshape(S, B, D)
    change_sm = change_rows.reshape(S, B, D)
    diffrow_sm = diff_rows.reshape(S, B, D)
    lo_r = _plane(np_lo.reshape(S, B).T)
    na_r = _plane(nab).astype(jnp.float32)
    np_r = _plane(npb)
    ns_r = _plane(nsb)

    def tT(w):
        return jnp.transpose(w)

    consts = [
        tT(ans_embed), tT(time_embed), tT(ls_state),
        tT(pro_state0[0:1]), tT(skill_state0[0:1]),
        tT(W_pf[:D]), tT(W_pf[D:]), b_pf.reshape(D, 1),
        tT(W_sf[:D]), tT(W_sf[D:]), b_sf.reshape(D, 1),
        tT(W_af[:D]), tT(W_af[D:]), b_af.reshape(D, 1),
        tT(W_ps[:D]), tT(W_ps[D:]), b_ps.reshape(D, 1),
        tT(W_ss[:D]), tT(W_ss[D:]), b_ss.reshape(D, 1),
        tT(W_as[:D]), tT(W_as[D:]), b_as.reshape(D, 1),
        tT(W_out1[0:D]), tT(W_out1[D:2 * D]), tT(W_out1[2 * D:3 * D]),
        tT(W_out1[3 * D:]), b_out1.reshape(D, 1), W_out2, b_out2.reshape(1, 1),
    ]
    out = _run_scan(pro_sm, skill_sm, change_sm, diffrow_sm, lo_r, na_r,
                    np_r, ns_r, consts)
    return out.transpose(0, 2, 1).reshape(B, S)
